# baseline (speedup 1.0000x reference)
import functools

import jax
import jax.numpy as jnp
from jax import lax
from jax.experimental import pallas as pl
from jax.experimental.pallas import tpu as pltpu
from jax.experimental.pallas import tpu_sc as plsc

N_NODES = 10000
N_EDGES = 320000
NC = 2
NS = 16
NW = NC * NS
IDX_W = 128
E_PAD = 327680
IDX_ROWS = E_PAD // IDX_W
K_PER_W = IDX_ROWS // NW
N_PAD = 10112
ROWS_PER_TILE = N_PAD // NS
TRASH_ROW = N_NODES


def _vector_mesh():
    return plsc.VectorSubcoreMesh(core_axis_name="c", subcore_axis_name="s")


def _deg_partials(dst2d, zeros16, ones, width=16, tc_tiling=False):
    cp = pltpu.CompilerParams(use_tc_tiling_on_sc=tc_tiling)

    @functools.partial(
        pl.kernel,
        out_type=jax.ShapeDtypeStruct((2 * N_PAD, width), jnp.float32),
        mesh=_vector_mesh(),
        compiler_params=cp,
        scratch_types=[
            pltpu.VMEM((K_PER_W, IDX_W), jnp.int32),
            pltpu.VMEM((IDX_W, width), jnp.float32),
            pltpu.VMEM_SHARED((N_PAD, width), jnp.float32),
        ],
    )
    def k(dst_hbm, zeros_hbm, ones_hbm, out_hbm, dst_v, ones_v, acc):
        cid = lax.axis_index("c")
        sid = lax.axis_index("s")
        wid = sid * NC + cid
        pltpu.sync_copy(
            zeros_hbm.at[pl.ds(sid * ROWS_PER_TILE, ROWS_PER_TILE)],
            acc.at[pl.ds(sid * ROWS_PER_TILE, ROWS_PER_TILE)],
        )
        pltpu.sync_copy(ones_hbm, ones_v)
        pltpu.sync_copy(dst_hbm.at[pl.ds(wid * K_PER_W, K_PER_W)], dst_v)
        plsc.subcore_barrier()

        @pl.loop(0, K_PER_W)
        def _(j):
            pltpu.sync_copy(ones_v, acc.at[dst_v.at[j]], add=True)

        plsc.subcore_barrier()
        pltpu.sync_copy(
            acc.at[pl.ds(sid * ROWS_PER_TILE, ROWS_PER_TILE)],
            out_hbm.at[pl.ds(cid * N_PAD + sid * ROWS_PER_TILE, ROWS_PER_TILE)],
        )

    return k(dst2d, zeros16, ones).reshape(2, N_PAD, width)


K_SPLIT = 40


def _agg_partials(y, src2d, dst2d, zeros, d, tc_tiling=True,
                  nbuf=2, t0=80, t1=80):
    assert t0 % K_SPLIT == 0 and t1 % K_SPLIT == 0
    assert t0 + t1 == 2 * K_PER_W
    k_split = K_SPLIT

    @functools.partial(
        pl.kernel,
        out_type=jax.ShapeDtypeStruct((2 * N_PAD, d), jnp.float32),
        mesh=_vector_mesh(),
        compiler_params=pltpu.CompilerParams(use_tc_tiling_on_sc=tc_tiling),
        scratch_types=[
            pltpu.VMEM((k_split, IDX_W), jnp.int32),
            pltpu.VMEM((k_split, IDX_W), jnp.int32),
        ] + [pltpu.VMEM((IDX_W, d), jnp.float32) for _ in range(nbuf)]
          + [pltpu.VMEM_SHARED((N_PAD, d), jnp.float32)]
          + [pltpu.SemaphoreType.DMA for _ in range(2 * nbuf)],
    )
    def k(y_hbm, src_hbm, dst_hbm, zeros_hbm, out_hbm,
          src_v, dst_v, *rest):
        rows = rest[:nbuf]
        acc = rest[nbuf]
        gsem = rest[nbuf + 1:nbuf + 1 + nbuf]
        ssem = rest[nbuf + 1 + nbuf:]
        cid = lax.axis_index("c")
        sid = lax.axis_index("s")
        wid = sid * NC + cid
        pltpu.sync_copy(
            zeros_hbm.at[pl.ds(sid * ROWS_PER_TILE, ROWS_PER_TILE)],
            acc.at[pl.ds(sid * ROWS_PER_TILE, ROWS_PER_TILE)],
        )

        def start_gather(b, j):
            pltpu.async_copy(y_hbm.at[src_v.at[j]], rows[b], gsem[b])

        def wait_gather(b):
            pltpu.make_async_copy(y_hbm.at[src_v.at[0]], rows[b],
                                  gsem[b]).wait()

        def start_scatter(b, j):
            pltpu.async_copy(rows[b], acc.at[dst_v.at[j]], ssem[b],
                             add=True)

        def wait_scatter(b):
            pltpu.make_async_copy(y_hbm.at[src_v.at[0]], rows[b],
                                  ssem[b]).wait()

        t_c = jnp.where(cid == 0, t0, t1)
        tile_base = cid * (16 * t0) + sid * t_c
        n_steps = t_c // K_SPLIT
        plsc.subcore_barrier()

        @pl.loop(0, n_steps)
        def _(h):
            base = tile_base + h * k_split
            pltpu.sync_copy(src_hbm.at[pl.ds(base, k_split)], src_v)
            pltpu.sync_copy(dst_hbm.at[pl.ds(base, k_split)], dst_v)

            for b in range(nbuf):
                start_gather(b, b)

            @pl.loop(0, k_split - nbuf, step=nbuf)
            def _(j):
                for b in range(nbuf):
                    wait_gather(b)
                    start_scatter(b, j + b)
                for b in range(nbuf):
                    wait_scatter(b)
                    start_gather(b, j + nbuf + b)

            for b in range(nbuf):
                wait_gather(b)
                start_scatter(b, k_split - nbuf + b)
            for b in range(nbuf):
                wait_scatter(b)

        plsc.subcore_barrier()
        pltpu.sync_copy(
            acc.at[pl.ds(sid * ROWS_PER_TILE, ROWS_PER_TILE)],
            out_hbm.at[pl.ds(cid * N_PAD + sid * ROWS_PER_TILE, ROWS_PER_TILE)],
        )

    return k(y, src2d, dst2d, zeros).reshape(2, N_PAD, d)


_R = 1000


def _tc_pre(x, w1, degp):
    def body(x_ref, w_ref, dp_ref, y_ref, dis_ref):
        deg = 1.0 + dp_ref[0, :, 0] + dp_ref[1, :, 0]
        dis = lax.rsqrt(deg)
        y = jnp.dot(x_ref[...], w_ref[...],
                    preferred_element_type=jnp.float32,
                    precision=lax.Precision.HIGHEST)
        y_ref[...] = y * dis[:, None]
        dis_ref[...] = jnp.broadcast_to(dis[:, None], (_R, 128))

    return pl.pallas_call(
        body,
        grid=(N_NODES // _R,),
        in_specs=[
            pl.BlockSpec((_R, 128), lambda i: (i, 0)),
            pl.BlockSpec((128, 128), lambda i: (0, 0)),
            pl.BlockSpec((2, _R, 16), lambda i: (0, i, 0)),
        ],
        out_specs=[
            pl.BlockSpec((_R, 128), lambda i: (i, 0)),
            pl.BlockSpec((_R, 128), lambda i: (i, 0)),
        ],
        out_shape=[
            jax.ShapeDtypeStruct((N_NODES, 128), jnp.float32),
            jax.ShapeDtypeStruct((N_NODES, 128), jnp.float32),
        ],
    )(x, w1, degp)


def _tc_mid(p, y1s, dis, b1, w2):
    def body(p_ref, y_ref, dis_ref, b_ref, w_ref, o_ref):
        agg = p_ref[0] + p_ref[1] + y_ref[...]
        h = jnp.maximum(dis_ref[...] * agg + b_ref[...], 0.0)
        y2 = jnp.dot(h, w_ref[...],
                     preferred_element_type=jnp.float32,
                     precision=lax.Precision.HIGHEST)
        o_ref[...] = y2 * dis_ref[...][:, :64]

    return pl.pallas_call(
        body,
        grid=(N_NODES // _R,),
        in_specs=[
            pl.BlockSpec((2, _R, 128), lambda i: (0, i, 0)),
            pl.BlockSpec((_R, 128), lambda i: (i, 0)),
            pl.BlockSpec((_R, 128), lambda i: (i, 0)),
            pl.BlockSpec((1, 128), lambda i: (0, 0)),
            pl.BlockSpec((128, 64), lambda i: (0, 0)),
        ],
        out_specs=pl.BlockSpec((_R, 64), lambda i: (i, 0)),
        out_shape=jax.ShapeDtypeStruct((N_NODES, 64), jnp.float32),
    )(p, y1s, dis, b1, w2)


def _tc_post(q, y2s, dis, b2):
    def body(q_ref, y_ref, dis_ref, b_ref, o_ref):
        agg = q_ref[0] + q_ref[1] + y_ref[...]
        o_ref[...] = dis_ref[...][:, :64] * agg + b_ref[...]

    return pl.pallas_call(
        body,
        grid=(N_NODES // _R,),
        in_specs=[
            pl.BlockSpec((2, _R, 64), lambda i: (0, i, 0)),
            pl.BlockSpec((_R, 64), lambda i: (i, 0)),
            pl.BlockSpec((_R, 128), lambda i: (i, 0)),
            pl.BlockSpec((1, 64), lambda i: (0, 0)),
        ],
        out_specs=pl.BlockSpec((_R, 64), lambda i: (i, 0)),
        out_shape=jax.ShapeDtypeStruct((N_NODES, 64), jnp.float32),
    )(q, y2s, dis, b2)


def kernel(x, edge_index, W1, b1, W2, b2):
    ei = edge_index.astype(jnp.int32)
    pad = E_PAD - N_EDGES
    src2d = jnp.concatenate(
        [ei[0], jnp.zeros((pad,), jnp.int32)]).reshape(IDX_ROWS, IDX_W)
    dst2d = jnp.concatenate(
        [ei[1], jnp.full((pad,), TRASH_ROW, jnp.int32)]).reshape(IDX_ROWS, IDX_W)

    zeros16 = jnp.zeros((N_PAD, 16), jnp.float32)
    zeros64 = jnp.zeros((N_PAD, 64), jnp.float32)
    zeros128 = jnp.zeros((N_PAD, 128), jnp.float32)
    ones = jnp.ones((IDX_W, 16), jnp.float32)

    degp = _deg_partials(dst2d, zeros16, ones)
    y1s, dis = _tc_pre(x, W1, degp)
    p = _agg_partials(y1s, src2d, dst2d, zeros128, 128,
                      tc_tiling=True, nbuf=2, t0=40, t1=120)
    y2s = _tc_mid(p, y1s, dis, b1.reshape(1, 128), W2)
    q = _agg_partials(y2s, src2d, dst2d, zeros64, 64,
                      tc_tiling=False, nbuf=4, t0=40, t1=120)
    return _tc_post(q, y2s, dis, b2.reshape(1, 64))


# speedup vs baseline: 1.0694x; 1.0694x over previous
import functools

import jax
import jax.numpy as jnp
from jax import lax
from jax.experimental import pallas as pl
from jax.experimental.pallas import tpu as pltpu
from jax.experimental.pallas import tpu_sc as plsc

N_NODES = 10000
N_EDGES = 320000
NC = 2
NS = 16
NW = NC * NS
IDX_W = 128
E_PAD = 327680
IDX_ROWS = E_PAD // IDX_W
K_PER_W = IDX_ROWS // NW
N_PAD = 10112
ROWS_PER_TILE = N_PAD // NS
TRASH_ROW = N_NODES


def _vector_mesh():
    return plsc.VectorSubcoreMesh(core_axis_name="c", subcore_axis_name="s")


def _deg_partials(dst2d, zeros16, ones, width=16, tc_tiling=False):
    cp = pltpu.CompilerParams(use_tc_tiling_on_sc=tc_tiling)

    @functools.partial(
        pl.kernel,
        out_type=jax.ShapeDtypeStruct((2 * N_PAD, width), jnp.float32),
        mesh=_vector_mesh(),
        compiler_params=cp,
        scratch_types=[
            pltpu.VMEM((K_PER_W, IDX_W), jnp.int32),
            pltpu.VMEM((IDX_W, width), jnp.float32),
            pltpu.VMEM_SHARED((N_PAD, width), jnp.float32),
        ],
    )
    def k(dst_hbm, zeros_hbm, ones_hbm, out_hbm, dst_v, ones_v, acc):
        cid = lax.axis_index("c")
        sid = lax.axis_index("s")
        wid = sid * NC + cid
        pltpu.sync_copy(
            zeros_hbm.at[pl.ds(sid * ROWS_PER_TILE, ROWS_PER_TILE)],
            acc.at[pl.ds(sid * ROWS_PER_TILE, ROWS_PER_TILE)],
        )
        pltpu.sync_copy(ones_hbm, ones_v)
        pltpu.sync_copy(dst_hbm.at[pl.ds(wid * K_PER_W, K_PER_W)], dst_v)
        plsc.subcore_barrier()

        @pl.loop(0, K_PER_W)
        def _(j):
            pltpu.sync_copy(ones_v, acc.at[dst_v.at[j]], add=True)

        plsc.subcore_barrier()
        pltpu.sync_copy(
            acc.at[pl.ds(sid * ROWS_PER_TILE, ROWS_PER_TILE)],
            out_hbm.at[pl.ds(cid * N_PAD + sid * ROWS_PER_TILE, ROWS_PER_TILE)],
        )

    return k(dst2d, zeros16, ones).reshape(2, N_PAD, width)


K_SPLIT = 40


def _agg_partials(y, src2d, dst2d, zeros, d, tc_tiling=True,
                  nbuf=2, t0=80, t1=80):
    assert t0 % K_SPLIT == 0 and t1 % K_SPLIT == 0
    assert t0 + t1 == 2 * K_PER_W
    k_split = K_SPLIT

    @functools.partial(
        pl.kernel,
        out_type=jax.ShapeDtypeStruct((2 * N_PAD, d), jnp.float32),
        mesh=_vector_mesh(),
        compiler_params=pltpu.CompilerParams(use_tc_tiling_on_sc=tc_tiling),
        scratch_types=[
            pltpu.VMEM((k_split, IDX_W), jnp.int32),
            pltpu.VMEM((k_split, IDX_W), jnp.int32),
        ] + [pltpu.VMEM((IDX_W, d), jnp.float32) for _ in range(nbuf)]
          + [pltpu.VMEM_SHARED((N_PAD, d), jnp.float32)]
          + [pltpu.SemaphoreType.DMA for _ in range(2 * nbuf)],
    )
    def k(y_hbm, src_hbm, dst_hbm, zeros_hbm, out_hbm,
          src_v, dst_v, *rest):
        rows = rest[:nbuf]
        acc = rest[nbuf]
        gsem = rest[nbuf + 1:nbuf + 1 + nbuf]
        ssem = rest[nbuf + 1 + nbuf:]
        cid = lax.axis_index("c")
        sid = lax.axis_index("s")
        wid = sid * NC + cid
        pltpu.sync_copy(
            zeros_hbm.at[pl.ds(sid * ROWS_PER_TILE, ROWS_PER_TILE)],
            acc.at[pl.ds(sid * ROWS_PER_TILE, ROWS_PER_TILE)],
        )

        def start_gather(b, j):
            pltpu.async_copy(y_hbm.at[src_v.at[j]], rows[b], gsem[b])

        def wait_gather(b):
            pltpu.make_async_copy(y_hbm.at[src_v.at[0]], rows[b],
                                  gsem[b]).wait()

        def start_scatter(b, j):
            pltpu.async_copy(rows[b], acc.at[dst_v.at[j]], ssem[b],
                             add=True)

        def wait_scatter(b):
            pltpu.make_async_copy(y_hbm.at[src_v.at[0]], rows[b],
                                  ssem[b]).wait()

        t_c = jnp.where(cid == 0, t0, t1)
        tile_base = cid * (16 * t0) + sid * t_c
        n_steps = t_c // K_SPLIT
        plsc.subcore_barrier()

        @pl.loop(0, n_steps)
        def _(h):
            base = tile_base + h * k_split
            pltpu.sync_copy(src_hbm.at[pl.ds(base, k_split)], src_v)
            pltpu.sync_copy(dst_hbm.at[pl.ds(base, k_split)], dst_v)

            for b in range(nbuf):
                start_gather(b, b)

            @pl.loop(0, k_split - nbuf, step=nbuf)
            def _(j):
                for b in range(nbuf):
                    wait_gather(b)
                    start_scatter(b, j + b)
                for b in range(nbuf):
                    wait_scatter(b)
                    start_gather(b, j + nbuf + b)

            for b in range(nbuf):
                wait_gather(b)
                start_scatter(b, k_split - nbuf + b)
            for b in range(nbuf):
                wait_scatter(b)

        plsc.subcore_barrier()
        pltpu.sync_copy(
            acc.at[pl.ds(sid * ROWS_PER_TILE, ROWS_PER_TILE)],
            out_hbm.at[pl.ds(cid * N_PAD + sid * ROWS_PER_TILE, ROWS_PER_TILE)],
        )

    return k(y, src2d, dst2d, zeros).reshape(2, N_PAD, d)


_R = 1000


def _tc_pre(x, w1, degp):
    def body(x_ref, w_ref, dp_ref, y_ref, dis_ref):
        deg = 1.0 + dp_ref[0, :, 0] + dp_ref[1, :, 0]
        dis = lax.rsqrt(deg)
        y = jnp.dot(x_ref[...], w_ref[...],
                    preferred_element_type=jnp.float32,
                    precision=lax.Precision.HIGHEST)
        y_ref[...] = y * dis[:, None]
        dis_ref[...] = jnp.broadcast_to(dis[:, None], (_R, 128))

    return pl.pallas_call(
        body,
        grid=(N_NODES // _R,),
        in_specs=[
            pl.BlockSpec((_R, 128), lambda i: (i, 0)),
            pl.BlockSpec((128, 128), lambda i: (0, 0)),
            pl.BlockSpec((2, _R, 16), lambda i: (0, i, 0)),
        ],
        out_specs=[
            pl.BlockSpec((_R, 128), lambda i: (i, 0)),
            pl.BlockSpec((_R, 128), lambda i: (i, 0)),
        ],
        out_shape=[
            jax.ShapeDtypeStruct((N_NODES, 128), jnp.float32),
            jax.ShapeDtypeStruct((N_NODES, 128), jnp.float32),
        ],
    )(x, w1, degp)


def _tc_mid(p, y1s, dis, b1, w2):
    def body(p_ref, y_ref, dis_ref, b_ref, w_ref, o_ref):
        agg = p_ref[0] + p_ref[1] + y_ref[...]
        h = jnp.maximum(dis_ref[...] * agg + b_ref[...], 0.0)
        y2 = jnp.dot(h, w_ref[...],
                     preferred_element_type=jnp.float32,
                     precision=lax.Precision.HIGHEST)
        o_ref[...] = y2 * dis_ref[...][:, :64]

    return pl.pallas_call(
        body,
        grid=(N_NODES // _R,),
        in_specs=[
            pl.BlockSpec((2, _R, 128), lambda i: (0, i, 0)),
            pl.BlockSpec((_R, 128), lambda i: (i, 0)),
            pl.BlockSpec((_R, 128), lambda i: (i, 0)),
            pl.BlockSpec((1, 128), lambda i: (0, 0)),
            pl.BlockSpec((128, 64), lambda i: (0, 0)),
        ],
        out_specs=pl.BlockSpec((_R, 64), lambda i: (i, 0)),
        out_shape=jax.ShapeDtypeStruct((N_NODES, 64), jnp.float32),
    )(p, y1s, dis, b1, w2)


def _tc_post(q, y2s, dis, b2):
    def body(q_ref, y_ref, dis_ref, b_ref, o_ref):
        agg = q_ref[0] + q_ref[1] + y_ref[...]
        o_ref[...] = dis_ref[...][:, :64] * agg + b_ref[...]

    return pl.pallas_call(
        body,
        grid=(N_NODES // _R,),
        in_specs=[
            pl.BlockSpec((2, _R, 64), lambda i: (0, i, 0)),
            pl.BlockSpec((_R, 64), lambda i: (i, 0)),
            pl.BlockSpec((_R, 128), lambda i: (i, 0)),
            pl.BlockSpec((1, 64), lambda i: (0, 0)),
        ],
        out_specs=pl.BlockSpec((_R, 64), lambda i: (i, 0)),
        out_shape=jax.ShapeDtypeStruct((N_NODES, 64), jnp.float32),
    )(q, y2s, dis, b2)


def kernel(x, edge_index, W1, b1, W2, b2):
    ei = edge_index.astype(jnp.int32)
    pad = E_PAD - N_EDGES
    src2d = jnp.concatenate(
        [ei[0], jnp.zeros((pad,), jnp.int32)]).reshape(IDX_ROWS, IDX_W)
    dst2d = jnp.concatenate(
        [ei[1], jnp.full((pad,), TRASH_ROW, jnp.int32)]).reshape(IDX_ROWS, IDX_W)

    zeros16 = jnp.zeros((N_PAD, 16), jnp.float32)
    zeros64 = jnp.zeros((N_PAD, 64), jnp.float32)
    zeros128 = jnp.zeros((N_PAD, 128), jnp.float32)
    ones = jnp.ones((IDX_W, 16), jnp.float32)

    degp = _deg_partials(dst2d, zeros16, ones)
    y1s, dis = _tc_pre(x, W1, degp)
    p = _agg_partials(y1s, src2d, dst2d, zeros128, 128,
                      tc_tiling=True, nbuf=2, t0=80, t1=80)
    y2s = _tc_mid(p, y1s, dis, b1.reshape(1, 128), W2)
    q = _agg_partials(y2s, src2d, dst2d, zeros64, 64,
                      tc_tiling=False, nbuf=4, t0=80, t1=80)
    return _tc_post(q, y2s, dis, b2.reshape(1, 64))


# speedup vs baseline: 1.3322x; 1.2458x over previous
import functools

import jax
import jax.numpy as jnp
from jax import lax
from jax.experimental import pallas as pl
from jax.experimental.pallas import tpu as pltpu
from jax.experimental.pallas import tpu_sc as plsc

N_NODES = 10000
N_EDGES = 320000
NC = 2
NS = 16
NW = NC * NS
IDX_W = 128
E_PAD = 327680
IDX_ROWS = E_PAD // IDX_W
K_PER_W = IDX_ROWS // NW
N_PAD = 10112
ROWS_PER_TILE = N_PAD // NS
TRASH_ROW = N_NODES


def _vector_mesh():
    return plsc.VectorSubcoreMesh(core_axis_name="c", subcore_axis_name="s")


def _deg_partials(dst2d, zeros16, ones, width=16, tc_tiling=False):
    cp = pltpu.CompilerParams(use_tc_tiling_on_sc=tc_tiling)

    @functools.partial(
        pl.kernel,
        out_type=jax.ShapeDtypeStruct((2 * N_PAD, width), jnp.float32),
        mesh=_vector_mesh(),
        compiler_params=cp,
        scratch_types=[
            pltpu.VMEM((K_PER_W, IDX_W), jnp.int32),
            pltpu.VMEM((IDX_W, width), jnp.float32),
            pltpu.VMEM_SHARED((N_PAD, width), jnp.float32),
        ],
    )
    def k(dst_hbm, zeros_hbm, ones_hbm, out_hbm, dst_v, ones_v, acc):
        cid = lax.axis_index("c")
        sid = lax.axis_index("s")
        wid = sid * NC + cid
        pltpu.sync_copy(
            zeros_hbm.at[pl.ds(sid * ROWS_PER_TILE, ROWS_PER_TILE)],
            acc.at[pl.ds(sid * ROWS_PER_TILE, ROWS_PER_TILE)],
        )
        pltpu.sync_copy(ones_hbm, ones_v)
        pltpu.sync_copy(dst_hbm.at[pl.ds(wid * K_PER_W, K_PER_W)], dst_v)
        plsc.subcore_barrier()

        @pl.loop(0, K_PER_W)
        def _(j):
            pltpu.sync_copy(ones_v, acc.at[dst_v.at[j]], add=True)

        plsc.subcore_barrier()
        pltpu.sync_copy(
            acc.at[pl.ds(sid * ROWS_PER_TILE, ROWS_PER_TILE)],
            out_hbm.at[pl.ds(cid * N_PAD + sid * ROWS_PER_TILE, ROWS_PER_TILE)],
        )

    return k(dst2d, zeros16, ones).reshape(2, N_PAD, width)


K_SPLIT = 40


TBL_PER_TILE = N_NODES // NS


def _agg_partials(y, src2d, dst2d, zeros, d, tc_tiling=True,
                  nbuf=2, t0=80, t1=80, spmem_table=False):
    assert t0 % K_SPLIT == 0 and t1 % K_SPLIT == 0
    assert t0 + t1 == 2 * K_PER_W
    k_split = K_SPLIT

    @functools.partial(
        pl.kernel,
        out_type=jax.ShapeDtypeStruct((2 * N_PAD, d), jnp.float32),
        mesh=_vector_mesh(),
        compiler_params=pltpu.CompilerParams(use_tc_tiling_on_sc=tc_tiling),
        scratch_types=[
            pltpu.VMEM((k_split, IDX_W), jnp.int32),
            pltpu.VMEM((k_split, IDX_W), jnp.int32),
        ] + [pltpu.VMEM((IDX_W, d), jnp.float32) for _ in range(nbuf)]
          + [pltpu.VMEM_SHARED((N_PAD, d), jnp.float32)]
          + ([pltpu.VMEM_SHARED((N_NODES, d), jnp.float32)]
             if spmem_table else [])
          + [pltpu.SemaphoreType.DMA for _ in range(2 * nbuf)],
    )
    def k(y_hbm, src_hbm, dst_hbm, zeros_hbm, out_hbm,
          src_v, dst_v, *rest):
        rows = rest[:nbuf]
        acc = rest[nbuf]
        ntbl = 1 if spmem_table else 0
        tbl = rest[nbuf + 1] if spmem_table else y_hbm
        gsem = rest[nbuf + 1 + ntbl:nbuf + 1 + ntbl + nbuf]
        ssem = rest[nbuf + 1 + ntbl + nbuf:]
        cid = lax.axis_index("c")
        sid = lax.axis_index("s")
        pltpu.sync_copy(
            zeros_hbm.at[pl.ds(sid * ROWS_PER_TILE, ROWS_PER_TILE)],
            acc.at[pl.ds(sid * ROWS_PER_TILE, ROWS_PER_TILE)],
        )
        if spmem_table:
            pltpu.sync_copy(
                y_hbm.at[pl.ds(sid * TBL_PER_TILE, TBL_PER_TILE)],
                tbl.at[pl.ds(sid * TBL_PER_TILE, TBL_PER_TILE)],
            )

        def start_gather(b, j):
            pltpu.async_copy(tbl.at[src_v.at[j]], rows[b], gsem[b])

        def wait_gather(b):
            pltpu.make_async_copy(tbl.at[src_v.at[0]], rows[b],
                                  gsem[b]).wait()

        def start_scatter(b, j):
            pltpu.async_copy(rows[b], acc.at[dst_v.at[j]], ssem[b],
                             add=True)

        def wait_scatter(b):
            pltpu.make_async_copy(y_hbm.at[src_v.at[0]], rows[b],
                                  ssem[b]).wait()

        t_c = jnp.where(cid == 0, t0, t1)
        tile_base = cid * (16 * t0) + sid * t_c
        n_steps = t_c // K_SPLIT
        plsc.subcore_barrier()

        @pl.loop(0, n_steps)
        def _(h):
            base = tile_base + h * k_split
            pltpu.sync_copy(src_hbm.at[pl.ds(base, k_split)], src_v)
            pltpu.sync_copy(dst_hbm.at[pl.ds(base, k_split)], dst_v)

            for b in range(nbuf):
                start_gather(b, b)

            @pl.loop(0, k_split - nbuf, step=nbuf)
            def _(j):
                for b in range(nbuf):
                    wait_gather(b)
                    start_scatter(b, j + b)
                for b in range(nbuf):
                    wait_scatter(b)
                    start_gather(b, j + nbuf + b)

            for b in range(nbuf):
                wait_gather(b)
                start_scatter(b, k_split - nbuf + b)
            for b in range(nbuf):
                wait_scatter(b)

        plsc.subcore_barrier()
        pltpu.sync_copy(
            acc.at[pl.ds(sid * ROWS_PER_TILE, ROWS_PER_TILE)],
            out_hbm.at[pl.ds(cid * N_PAD + sid * ROWS_PER_TILE, ROWS_PER_TILE)],
        )

    return k(y, src2d, dst2d, zeros).reshape(2, N_PAD, d)


_R = 1000


def _tc_pre(x, w1, degp):
    def body(x_ref, w_ref, dp_ref, y_ref, dis_ref):
        deg = 1.0 + dp_ref[0, :, 0] + dp_ref[1, :, 0]
        dis = lax.rsqrt(deg)
        y = jnp.dot(x_ref[...], w_ref[...],
                    preferred_element_type=jnp.float32,
                    precision=lax.Precision.HIGHEST)
        y_ref[...] = y * dis[:, None]
        dis_ref[...] = jnp.broadcast_to(dis[:, None], (_R, 128))

    return pl.pallas_call(
        body,
        grid=(N_NODES // _R,),
        in_specs=[
            pl.BlockSpec((_R, 128), lambda i: (i, 0)),
            pl.BlockSpec((128, 128), lambda i: (0, 0)),
            pl.BlockSpec((2, _R, 16), lambda i: (0, i, 0)),
        ],
        out_specs=[
            pl.BlockSpec((_R, 128), lambda i: (i, 0)),
            pl.BlockSpec((_R, 128), lambda i: (i, 0)),
        ],
        out_shape=[
            jax.ShapeDtypeStruct((N_NODES, 128), jnp.float32),
            jax.ShapeDtypeStruct((N_NODES, 128), jnp.float32),
        ],
    )(x, w1, degp)


def _tc_mid(p, y1s, dis, b1, w2):
    def body(p_ref, y_ref, dis_ref, b_ref, w_ref, o_ref):
        agg = p_ref[0] + p_ref[1] + y_ref[...]
        h = jnp.maximum(dis_ref[...] * agg + b_ref[...], 0.0)
        y2 = jnp.dot(h, w_ref[...],
                     preferred_element_type=jnp.float32,
                     precision=lax.Precision.HIGHEST)
        o_ref[...] = y2 * dis_ref[...][:, :64]

    return pl.pallas_call(
        body,
        grid=(N_NODES // _R,),
        in_specs=[
            pl.BlockSpec((2, _R, 128), lambda i: (0, i, 0)),
            pl.BlockSpec((_R, 128), lambda i: (i, 0)),
            pl.BlockSpec((_R, 128), lambda i: (i, 0)),
            pl.BlockSpec((1, 128), lambda i: (0, 0)),
            pl.BlockSpec((128, 64), lambda i: (0, 0)),
        ],
        out_specs=pl.BlockSpec((_R, 64), lambda i: (i, 0)),
        out_shape=jax.ShapeDtypeStruct((N_NODES, 64), jnp.float32),
    )(p, y1s, dis, b1, w2)


def _tc_post(q, y2s, dis, b2):
    def body(q_ref, y_ref, dis_ref, b_ref, o_ref):
        agg = q_ref[0] + q_ref[1] + y_ref[...]
        o_ref[...] = dis_ref[...][:, :64] * agg + b_ref[...]

    return pl.pallas_call(
        body,
        grid=(N_NODES // _R,),
        in_specs=[
            pl.BlockSpec((2, _R, 64), lambda i: (0, i, 0)),
            pl.BlockSpec((_R, 64), lambda i: (i, 0)),
            pl.BlockSpec((_R, 128), lambda i: (i, 0)),
            pl.BlockSpec((1, 64), lambda i: (0, 0)),
        ],
        out_specs=pl.BlockSpec((_R, 64), lambda i: (i, 0)),
        out_shape=jax.ShapeDtypeStruct((N_NODES, 64), jnp.float32),
    )(q, y2s, dis, b2)


def kernel(x, edge_index, W1, b1, W2, b2):
    ei = edge_index.astype(jnp.int32)
    pad = E_PAD - N_EDGES
    src2d = jnp.concatenate(
        [ei[0], jnp.zeros((pad,), jnp.int32)]).reshape(IDX_ROWS, IDX_W)
    dst2d = jnp.concatenate(
        [ei[1], jnp.full((pad,), TRASH_ROW, jnp.int32)]).reshape(IDX_ROWS, IDX_W)

    zeros16 = jnp.zeros((N_PAD, 16), jnp.float32)
    zeros64 = jnp.zeros((N_PAD, 64), jnp.float32)
    zeros128 = jnp.zeros((N_PAD, 128), jnp.float32)
    ones = jnp.ones((IDX_W, 16), jnp.float32)

    degp = _deg_partials(dst2d, zeros16, ones)
    y1s, dis = _tc_pre(x, W1, degp)
    p = _agg_partials(y1s, src2d, dst2d, zeros128, 128,
                      tc_tiling=True, nbuf=2, t0=80, t1=80)
    y2s = _tc_mid(p, y1s, dis, b1.reshape(1, 128), W2)
    q = _agg_partials(y2s, src2d, dst2d, zeros64, 64,
                      tc_tiling=False, nbuf=4, t0=80, t1=80, spmem_table=True)
    return _tc_post(q, y2s, dis, b2.reshape(1, 64))


# speedup vs baseline: 2.2986x; 1.7255x over previous
import functools

import jax
import jax.numpy as jnp
from jax import lax
from jax.experimental import pallas as pl
from jax.experimental.pallas import tpu as pltpu
from jax.experimental.pallas import tpu_sc as plsc

N_NODES = 10000
N_EDGES = 320000
NC = 2
NS = 16
NW = NC * NS
IDX_W = 128
E_PAD = 327680
IDX_ROWS = E_PAD // IDX_W
K_PER_W = IDX_ROWS // NW
N_PAD = 10112
ROWS_PER_TILE = N_PAD // NS
TRASH_ROW = N_NODES


def _vector_mesh():
    return plsc.VectorSubcoreMesh(core_axis_name="c", subcore_axis_name="s")


def _deg_partials(dst2d, zeros16, ones, width=16, tc_tiling=False):
    cp = pltpu.CompilerParams(use_tc_tiling_on_sc=tc_tiling)

    @functools.partial(
        pl.kernel,
        out_type=jax.ShapeDtypeStruct((2 * N_PAD, width), jnp.float32),
        mesh=_vector_mesh(),
        compiler_params=cp,
        scratch_types=[
            pltpu.VMEM((K_PER_W, IDX_W), jnp.int32),
            pltpu.VMEM((IDX_W, width), jnp.float32),
            pltpu.VMEM_SHARED((N_PAD, width), jnp.float32),
        ],
    )
    def k(dst_hbm, zeros_hbm, ones_hbm, out_hbm, dst_v, ones_v, acc):
        cid = lax.axis_index("c")
        sid = lax.axis_index("s")
        wid = sid * NC + cid
        pltpu.sync_copy(
            zeros_hbm.at[pl.ds(sid * ROWS_PER_TILE, ROWS_PER_TILE)],
            acc.at[pl.ds(sid * ROWS_PER_TILE, ROWS_PER_TILE)],
        )
        pltpu.sync_copy(ones_hbm, ones_v)
        pltpu.sync_copy(dst_hbm.at[pl.ds(wid * K_PER_W, K_PER_W)], dst_v)
        plsc.subcore_barrier()

        @pl.loop(0, K_PER_W)
        def _(j):
            pltpu.sync_copy(ones_v, acc.at[dst_v.at[j]], add=True)

        plsc.subcore_barrier()
        pltpu.sync_copy(
            acc.at[pl.ds(sid * ROWS_PER_TILE, ROWS_PER_TILE)],
            out_hbm.at[pl.ds(cid * N_PAD + sid * ROWS_PER_TILE, ROWS_PER_TILE)],
        )

    return k(dst2d, zeros16, ones).reshape(2, N_PAD, width)


K_SPLIT = 40


TBL_PER_TILE = N_NODES // NS


def _agg_partials(y, src2d, dst2d, zeros, d, tc_tiling=True,
                  nbuf=2, t0=80, t1=80, spmem_table=False, colsplit=False):
    assert t0 % K_SPLIT == 0 and t1 % K_SPLIT == 0
    assert t0 + t1 == 2 * K_PER_W
    k_split = K_SPLIT

    @functools.partial(
        pl.kernel,
        out_type=jax.ShapeDtypeStruct((2 * N_PAD, d), jnp.float32),
        mesh=_vector_mesh(),
        compiler_params=pltpu.CompilerParams(use_tc_tiling_on_sc=tc_tiling),
        scratch_types=[
            pltpu.VMEM((k_split, IDX_W), jnp.int32),
            pltpu.VMEM((k_split, IDX_W), jnp.int32),
        ] + [pltpu.VMEM((IDX_W, d), jnp.float32) for _ in range(nbuf)]
          + [pltpu.VMEM_SHARED((N_PAD, d), jnp.float32)]
          + ([pltpu.VMEM_SHARED((N_NODES, d), jnp.float32)]
             if spmem_table else [])
          + [pltpu.SemaphoreType.DMA for _ in range(2 * nbuf)],
    )
    def k(y_hbm, src_hbm, dst_hbm, zeros_hbm, out_hbm,
          src_v, dst_v, *rest):
        rows = rest[:nbuf]
        acc = rest[nbuf]
        ntbl = 1 if spmem_table else 0
        tbl = rest[nbuf + 1] if spmem_table else y_hbm
        gsem = rest[nbuf + 1 + ntbl:nbuf + 1 + ntbl + nbuf]
        ssem = rest[nbuf + 1 + ntbl + nbuf:]
        cid = lax.axis_index("c")
        sid = lax.axis_index("s")
        pltpu.sync_copy(
            zeros_hbm.at[pl.ds(sid * ROWS_PER_TILE, ROWS_PER_TILE)],
            acc.at[pl.ds(sid * ROWS_PER_TILE, ROWS_PER_TILE)],
        )
        if spmem_table:
            row_off = cid * N_NODES if colsplit else 0
            pltpu.sync_copy(
                y_hbm.at[pl.ds(row_off + sid * TBL_PER_TILE, TBL_PER_TILE)],
                tbl.at[pl.ds(sid * TBL_PER_TILE, TBL_PER_TILE)],
            )

        def start_gather(b, j):
            pltpu.async_copy(tbl.at[src_v.at[j]], rows[b], gsem[b])

        def wait_gather(b):
            pltpu.make_async_copy(tbl.at[src_v.at[0]], rows[b],
                                  gsem[b]).wait()

        def start_scatter(b, j):
            pltpu.async_copy(rows[b], acc.at[dst_v.at[j]], ssem[b],
                             add=True)

        def wait_scatter(b):
            pltpu.make_async_copy(y_hbm.at[src_v.at[0]], rows[b],
                                  ssem[b]).wait()

        if colsplit:
            t_c = 2 * K_PER_W
            tile_base = sid * t_c
        else:
            t_c = jnp.where(cid == 0, t0, t1)
            tile_base = cid * (16 * t0) + sid * t_c
        n_steps = t_c // K_SPLIT
        plsc.subcore_barrier()

        @pl.loop(0, n_steps)
        def _(h):
            base = tile_base + h * k_split
            pltpu.sync_copy(src_hbm.at[pl.ds(base, k_split)], src_v)
            pltpu.sync_copy(dst_hbm.at[pl.ds(base, k_split)], dst_v)

            for b in range(nbuf):
                start_gather(b, b)

            @pl.loop(0, k_split - nbuf, step=nbuf)
            def _(j):
                for b in range(nbuf):
                    wait_gather(b)
                    start_scatter(b, j + b)
                for b in range(nbuf):
                    wait_scatter(b)
                    start_gather(b, j + nbuf + b)

            for b in range(nbuf):
                wait_gather(b)
                start_scatter(b, k_split - nbuf + b)
            for b in range(nbuf):
                wait_scatter(b)

        plsc.subcore_barrier()
        pltpu.sync_copy(
            acc.at[pl.ds(sid * ROWS_PER_TILE, ROWS_PER_TILE)],
            out_hbm.at[pl.ds(cid * N_PAD + sid * ROWS_PER_TILE, ROWS_PER_TILE)],
        )

    return k(y, src2d, dst2d, zeros).reshape(2, N_PAD, d)


_R = 1000


def _tc_pre(x, w1, degp):
    def body(x_ref, w_ref, dp_ref, ya_ref, yb_ref, dis_ref):
        deg = 1.0 + dp_ref[0, :, 0] + dp_ref[1, :, 0]
        dis = lax.rsqrt(deg)
        y = jnp.dot(x_ref[...], w_ref[...],
                    preferred_element_type=jnp.float32,
                    precision=lax.Precision.HIGHEST)
        ys = y * dis[:, None]
        ya_ref[...] = ys[:, :64]
        yb_ref[...] = ys[:, 64:]
        dis_ref[...] = jnp.broadcast_to(dis[:, None], (_R, 128))

    return pl.pallas_call(
        body,
        grid=(N_NODES // _R,),
        in_specs=[
            pl.BlockSpec((_R, 128), lambda i: (i, 0)),
            pl.BlockSpec((128, 128), lambda i: (0, 0)),
            pl.BlockSpec((2, _R, 16), lambda i: (0, i, 0)),
        ],
        out_specs=[
            pl.BlockSpec((_R, 64), lambda i: (i, 0)),
            pl.BlockSpec((_R, 64), lambda i: (i, 0)),
            pl.BlockSpec((_R, 128), lambda i: (i, 0)),
        ],
        out_shape=[
            jax.ShapeDtypeStruct((N_NODES, 64), jnp.float32),
            jax.ShapeDtypeStruct((N_NODES, 64), jnp.float32),
            jax.ShapeDtypeStruct((N_NODES, 128), jnp.float32),
        ],
    )(x, w1, degp)


def _tc_mid(p, ya, yb, dis, b1, w2):
    def body(p_ref, ya_ref, yb_ref, dis_ref, b_ref, w_ref, o_ref):
        agg = jnp.concatenate(
            [p_ref[0] + ya_ref[...], p_ref[1] + yb_ref[...]], axis=1)
        h = jnp.maximum(dis_ref[...] * agg + b_ref[...], 0.0)
        y2 = jnp.dot(h, w_ref[...],
                     preferred_element_type=jnp.float32,
                     precision=lax.Precision.HIGHEST)
        o_ref[...] = y2 * dis_ref[...][:, :64]

    return pl.pallas_call(
        body,
        grid=(N_NODES // _R,),
        in_specs=[
            pl.BlockSpec((2, _R, 64), lambda i: (0, i, 0)),
            pl.BlockSpec((_R, 64), lambda i: (i, 0)),
            pl.BlockSpec((_R, 64), lambda i: (i, 0)),
            pl.BlockSpec((_R, 128), lambda i: (i, 0)),
            pl.BlockSpec((1, 128), lambda i: (0, 0)),
            pl.BlockSpec((128, 64), lambda i: (0, 0)),
        ],
        out_specs=pl.BlockSpec((_R, 64), lambda i: (i, 0)),
        out_shape=jax.ShapeDtypeStruct((N_NODES, 64), jnp.float32),
    )(p, ya, yb, dis, b1, w2)


def _tc_post(q, y2s, dis, b2):
    def body(q_ref, y_ref, dis_ref, b_ref, o_ref):
        agg = q_ref[0] + q_ref[1] + y_ref[...]
        o_ref[...] = dis_ref[...][:, :64] * agg + b_ref[...]

    return pl.pallas_call(
        body,
        grid=(N_NODES // _R,),
        in_specs=[
            pl.BlockSpec((2, _R, 64), lambda i: (0, i, 0)),
            pl.BlockSpec((_R, 64), lambda i: (i, 0)),
            pl.BlockSpec((_R, 128), lambda i: (i, 0)),
            pl.BlockSpec((1, 64), lambda i: (0, 0)),
        ],
        out_specs=pl.BlockSpec((_R, 64), lambda i: (i, 0)),
        out_shape=jax.ShapeDtypeStruct((N_NODES, 64), jnp.float32),
    )(q, y2s, dis, b2)


def kernel(x, edge_index, W1, b1, W2, b2):
    ei = edge_index.astype(jnp.int32)
    pad = E_PAD - N_EDGES
    src2d = jnp.concatenate(
        [ei[0], jnp.zeros((pad,), jnp.int32)]).reshape(IDX_ROWS, IDX_W)
    dst2d = jnp.concatenate(
        [ei[1], jnp.full((pad,), TRASH_ROW, jnp.int32)]).reshape(IDX_ROWS, IDX_W)

    zeros16 = jnp.zeros((N_PAD, 16), jnp.float32)
    zeros64 = jnp.zeros((N_PAD, 64), jnp.float32)
    ones = jnp.ones((IDX_W, 16), jnp.float32)

    degp = _deg_partials(dst2d, zeros16, ones)
    ya, yb, dis = _tc_pre(x, W1, degp)
    ycat = jnp.concatenate([ya, yb], axis=0)
    p = _agg_partials(ycat, src2d, dst2d, zeros64, 64,
                      tc_tiling=False, nbuf=4, spmem_table=True,
                      colsplit=True)
    y2s = _tc_mid(p, ya, yb, dis, b1.reshape(1, 128), W2)
    q = _agg_partials(y2s, src2d, dst2d, zeros64, 64,
                      tc_tiling=False, nbuf=4, t0=80, t1=80, spmem_table=True)
    return _tc_post(q, y2s, dis, b2.reshape(1, 64))
